# baseline (device time: 12630 ns/iter reference)
import jax
import jax.numpy as jnp
from jax import lax
from jax.experimental import pallas as pl
from jax.experimental.pallas import tpu as pltpu

N_DEV = 4
B = 2
SQ = 256
SKV = 256
HALO = 128
SFULL = SKV + 2 * HALO
HQ = 4
DH = 64
HD = HQ * DH
DM = 512
WINDOW = 128
SKV_GLOBAL = N_DEV * SKV


def kernel(x, Wq, K_ext, V_ext, Wo):
    K2 = K_ext.reshape(B, SKV, HD)
    V2 = V_ext.reshape(B, SKV, HD)

    def body(x_ref, wq_ref, k_ref, v_ref, wo_ref, out_ref,
             k_full, v_full, send_sems, recv_sems):
        my = lax.axis_index("i")
        left = (my + N_DEV - 1) % N_DEV
        right = (my + 1) % N_DEV

        barrier_sem = pltpu.get_barrier_semaphore()
        for nbr in (left, right):
            pl.semaphore_signal(
                barrier_sem, inc=1,
                device_id=(nbr,), device_id_type=pl.DeviceIdType.MESH,
            )
        pl.semaphore_wait(barrier_sem, 2)

        def halo_rdma(buf, src_off, dst_off, dev, sem_i):
            return pltpu.make_async_remote_copy(
                src_ref=buf.at[:, pl.ds(src_off, HALO), :],
                dst_ref=buf.at[:, pl.ds(dst_off, HALO), :],
                send_sem=send_sems.at[sem_i],
                recv_sem=recv_sems.at[sem_i],
                device_id=(dev,),
                device_id_type=pl.DeviceIdType.MESH,
            )

        k_full[:, HALO:HALO + SKV, :] = k_ref[...].astype(jnp.bfloat16)
        k_rdmas = [halo_rdma(k_full, HALO, SKV + HALO, left, 0),
                   halo_rdma(k_full, SKV, 0, right, 1)]
        for r in k_rdmas:
            r.start()
        v_full[:, HALO:HALO + SKV, :] = v_ref[...].astype(jnp.bfloat16)
        v_rdmas = [halo_rdma(v_full, HALO, SKV + HALO, left, 2),
                   halo_rdma(v_full, SKV, 0, right, 3)]
        for r in v_rdmas:
            r.start()

        wq = wq_ref[...].astype(jnp.bfloat16)
        wo = wo_ref[...].astype(jnp.bfloat16)
        x2 = x_ref[...].reshape(B * SQ, DM).astype(jnp.bfloat16)
        q_all = (lax.dot(x2, wq, preferred_element_type=jnp.float32)
                 * 0.125).astype(jnp.bfloat16)

        qi = lax.broadcasted_iota(jnp.int32, (SQ, SFULL), 0)
        kj = lax.broadcasted_iota(jnp.int32, (SQ, SFULL), 1)
        window = jnp.abs(qi - kj + HALO) <= WINDOW
        kg = kj + my * SKV - HALO
        mask = window & (kg >= 0) & (kg < SKV_GLOBAL)
        mbias = jnp.where(mask, 0.0, -1e9).astype(jnp.float32)
        b_l = mbias[:, 0:HALO]
        b_m = mbias[:, HALO:HALO + SKV]
        b_r = mbias[:, HALO + SKV:]

        def scores(qh, kblock, bias):
            s = lax.dot_general(
                qh, kblock, (((1,), (1,)), ((), ())),
                preferred_element_type=jnp.float32) + bias
            w = jnp.exp(s.astype(jnp.bfloat16))
            return w, jnp.sum(w, axis=-1, keepdims=True, dtype=jnp.float32)

        w_mid, d_mid, ctx_mid = [], [], []
        for b in range(B):
            k_own = k_full[b, HALO:HALO + SKV, :]
            v_own = v_full[b, HALO:HALO + SKV, :]
            for h in range(HQ):
                qh = q_all[b * SQ:(b + 1) * SQ, h * DH:(h + 1) * DH]
                w, d = scores(qh, k_own[:, h * DH:(h + 1) * DH], b_m)
                w_mid.append(w)
                d_mid.append(d)
                ctx_mid.append(lax.dot(
                    w, v_own[:, h * DH:(h + 1) * DH],
                    preferred_element_type=jnp.float32))

        for r in k_rdmas:
            r.wait()
        w_l, w_r, d_halo = [], [], []
        for b in range(B):
            k_lh = k_full[b, 0:HALO, :]
            k_rh = k_full[b, HALO + SKV:, :]
            for h in range(HQ):
                qh = q_all[b * SQ:(b + 1) * SQ, h * DH:(h + 1) * DH]
                wl, dl = scores(qh, k_lh[:, h * DH:(h + 1) * DH], b_l)
                wr, dr = scores(qh, k_rh[:, h * DH:(h + 1) * DH], b_r)
                w_l.append(wl)
                w_r.append(wr)
                d_halo.append(dl + dr)

        for r in v_rdmas:
            r.wait()
        ctxs = []
        for b in range(B):
            v_lh = v_full[b, 0:HALO, :]
            v_rh = v_full[b, HALO + SKV:, :]
            for h in range(HQ):
                i = b * HQ + h
                hs = slice(h * DH, (h + 1) * DH)
                ctx = (ctx_mid[i]
                       + lax.dot(w_l[i], v_lh[:, hs],
                                 preferred_element_type=jnp.float32)
                       + lax.dot(w_r[i], v_rh[:, hs],
                                 preferred_element_type=jnp.float32))
                ctxs.append(ctx / (d_mid[i] + d_halo[i]))
        ctx = jnp.concatenate(
            [jnp.concatenate(ctxs[b * HQ:(b + 1) * HQ], axis=1)
             for b in range(B)], axis=0)
        out = lax.dot(ctx.astype(jnp.bfloat16), wo,
                      preferred_element_type=jnp.float32)
        for b in range(B):
            out_ref[b] = out[b * SQ:(b + 1) * SQ, :]

    return pl.pallas_call(
        body,
        out_shape=jax.ShapeDtypeStruct((B, SQ, DM), jnp.float32),
        in_specs=[pl.BlockSpec(memory_space=pltpu.VMEM)] * 5,
        out_specs=pl.BlockSpec(memory_space=pltpu.VMEM),
        scratch_shapes=[
            pltpu.VMEM((B, SFULL, HD), jnp.bfloat16),
            pltpu.VMEM((B, SFULL, HD), jnp.bfloat16),
            pltpu.SemaphoreType.DMA((4,)),
            pltpu.SemaphoreType.DMA((4,)),
        ],
        compiler_params=pltpu.CompilerParams(collective_id=0),
    )(x, Wq, K2, V2, Wo)
